# baseline (speedup 1.0000x reference)
import jax
import jax.numpy as jnp
from jax import lax
from jax.experimental import pallas as pl
from jax.experimental.pallas import tpu as pltpu

_BN = 1024
_K = 4


def _proj_body(hl_ref, W2_ref, b2_ref, proj_ref):
    proj = lax.dot_general(hl_ref[...], W2_ref[...],
                           (((1,), (1,)), ((), ())),
                           precision=lax.Precision.HIGHEST,
                           preferred_element_type=jnp.float32)
    proj_ref[...] = proj + b2_ref[...]


def _stream_body(*refs):
    scal_ref, proj_ref, vW_ref = refs[:3]
    g_refs = refs[3:3 + _K]
    q_refs = refs[3 + _K:3 + 2 * _K]
    out_refs = refs[3 + 2 * _K:3 + 3 * _K]
    idx_ref = refs[3 + 3 * _K]
    m_ref, mi_ref = refs[3 + 3 * _K + 1:]

    i = pl.program_id(0)
    nb = pl.num_programs(0)

    temp = scal_ref[0, 0]
    tanh_c = scal_ref[0, 1]
    vb_c = scal_ref[0, 2]

    @pl.when(i == 0)
    def _init():
        m_ref[0] = -jnp.inf
        mi_ref[0] = 0

    for k in range(_K):
        hidden = jnp.tanh(q_refs[k][...] + proj_ref[...])
        col = jnp.sum(hidden * vW_ref[...], axis=1, keepdims=True)
        logits_blk = tanh_c * jnp.tanh((col + vb_c) / temp)
        out_refs[k][...] = logits_blk

        score = logits_blk + g_refs[k][...]
        local_max = jnp.max(score)
        ids = lax.broadcasted_iota(jnp.int32, score.shape, 0)
        local_arg = jnp.min(jnp.where(score == local_max, ids, score.shape[0]))
        base = (i * _K + k) * score.shape[0]
        cur_m = m_ref[0]
        upd = local_max > cur_m
        m_ref[0] = jnp.where(upd, local_max, cur_m)
        mi_ref[0] = jnp.where(upd, base + local_arg, mi_ref[0])

    @pl.when(i == nb - 1)
    def _fin():
        idx_ref[0, 0] = mi_ref[0]


def kernel(h, query, W2, b2, vW, vb, temperature, tanh_constant):
    N, L = query.shape
    bn = min(_BN, N)
    nsteps = N // (bn * _K)
    hl = h[-1].reshape(1, L)
    g = jax.random.gumbel(jax.random.key(42), (1, N), jnp.float32).reshape(N, 1)
    scal = jnp.stack([jnp.asarray(temperature, jnp.float32),
                      jnp.asarray(tanh_constant, jnp.float32),
                      vb.astype(jnp.float32)[0],
                      jnp.float32(0)]).reshape(1, 4)

    proj = pl.pallas_call(
        _proj_body,
        out_shape=jax.ShapeDtypeStruct((1, L), jnp.float32),
    )(hl, W2, b2.reshape(1, L))

    def blk_map(k):
        return lambda i: (i * _K + k, 0)

    outs = pl.pallas_call(
        _stream_body,
        grid=(nsteps,),
        in_specs=(
            [pl.BlockSpec(memory_space=pltpu.SMEM),
             pl.BlockSpec((1, L), lambda i: (0, 0)),
             pl.BlockSpec((1, L), lambda i: (0, 0))]
            + [pl.BlockSpec((bn, 1), blk_map(k)) for k in range(_K)]
            + [pl.BlockSpec((bn, L), blk_map(k)) for k in range(_K)]
        ),
        out_specs=(
            [pl.BlockSpec((bn, 1), lambda i: (i, 0)) for _ in range(_K)]
            + [pl.BlockSpec((1, 1), lambda i: (0, 0),
                            memory_space=pltpu.SMEM)]
        ),
        out_shape=(
            [jax.ShapeDtypeStruct((nsteps * bn, 1), jnp.float32)
             for _ in range(_K)]
            + [jax.ShapeDtypeStruct((1, 1), jnp.int32)]
        ),
        scratch_shapes=[
            pltpu.SMEM((1,), jnp.float32),
            pltpu.SMEM((1,), jnp.int32),
        ],
    )(scal, proj, vW, *([g] * _K), *([query] * _K))

    cols, idx = outs[:_K], outs[_K]
    logits = jnp.stack([c.reshape(nsteps, bn) for c in cols],
                       axis=1).reshape(1, N)
    return (idx[0, 0], logits)
